# baseline (speedup 1.0000x reference)
import functools

import jax
import jax.numpy as jnp
from jax import lax
from jax.experimental import pallas as pl
from jax.experimental.pallas import tpu as pltpu
from jax.experimental.pallas import tpu_sc as plsc

_N, _E, _D = 10000, 320000, 128
_NC, _NS = 2, 16
_NW = _NC * _NS
_CHUNK = 128
_GCHUNK = 256
_NSUB = _GCHUNK // _CHUNK
_N_PAD = 10112
_E_PER_W = 10240
_E_PAD = _E_PER_W * _NW
_N_ROUNDS = _E_PER_W // _GCHUNK
_ROWS_PER_TILE = _N_PAD // _NS
_SLAB_CHUNKS = []
_off = 0
while _off < _ROWS_PER_TILE:
    _n = min(_GCHUNK, _ROWS_PER_TILE - _off)
    _SLAB_CHUNKS.append((_off, _n))
    _off += _n


def _deg_body(dst_hbm, deg_out, dst_v, ones_v, deg_sh):
    c = lax.axis_index("c")
    s = lax.axis_index("s")
    wid = c * _NS + s

    zeros16 = jnp.zeros((16,), jnp.float32)
    ones16 = jnp.ones((16,), jnp.float32)

    def _fill_zeros(r, carry):
        for j in range(_D // 16):
            ones_v[r, pl.ds(j * 16, 16)] = zeros16
        return carry

    lax.fori_loop(0, _CHUNK, _fill_zeros, None)

    row0 = s * _ROWS_PER_TILE
    for off, n in _SLAB_CHUNKS:
        for o2 in range(0, n, _CHUNK):
            m = min(_CHUNK, n - o2)
            pltpu.sync_copy(ones_v.at[pl.ds(0, m)],
                            deg_sh.at[pl.ds(row0 + off + o2, m)])
    plsc.subcore_barrier()

    def _fill_ones(r, carry):
        for j in range(_D // 16):
            ones_v[r, pl.ds(j * 16, 16)] = ones16
        return carry

    lax.fori_loop(0, _CHUNK, _fill_ones, None)

    def _round(t, carry):
        base = pl.multiple_of(wid * _E_PER_W + t * _GCHUNK, 8)
        for q in range(_NSUB):
            pltpu.sync_copy(dst_hbm.at[pl.ds(base + q * _CHUNK, _CHUNK)],
                            dst_v.at[q])
        for q in range(_NSUB):
            pltpu.sync_copy(ones_v, deg_sh.at[dst_v.at[q]], add=True)
        return carry

    lax.fori_loop(0, _N_ROUNDS, _round, None)
    plsc.subcore_barrier()

    for off, n in _SLAB_CHUNKS:
        for o2 in range(0, n, _CHUNK):
            m = min(_CHUNK, n - o2)
            r = row0 + off + o2
            pltpu.sync_copy(deg_sh.at[pl.ds(r, m)], ones_v.at[pl.ds(0, m)])
            pltpu.sync_copy(ones_v.at[pl.ds(0, m)], deg_out.at[c, pl.ds(r, m)])


@functools.cache
def _sc_degree_call():
    return pl.kernel(
        _deg_body,
        out_type=jax.ShapeDtypeStruct((_NC, _N_PAD, _D), jnp.float32),
        mesh=plsc.VectorSubcoreMesh(core_axis_name="c", subcore_axis_name="s",
                                    num_cores=_NC, num_subcores=_NS),
        scratch_types=[
            pltpu.VMEM((_NSUB, _CHUNK), jnp.int32),
            pltpu.VMEM((_CHUNK, _D), jnp.float32),
            pltpu.VMEM_SHARED((_N_PAD, _D), jnp.float32),
        ],
    )


def _sc_body(h_hbm, src_hbm, dst_hbm, acc_out,
             src_v, dst_v, rows_v, acc_sh, gsem):
    c = lax.axis_index("c")
    s = lax.axis_index("s")
    wid = c * _NS + s

    zeros16 = jnp.zeros((16,), jnp.float32)

    def _fill_zeros(r, carry):
        for j in range(_D // 16):
            rows_v[r, pl.ds(j * 16, 16)] = zeros16
        return carry

    lax.fori_loop(0, _GCHUNK, _fill_zeros, None)

    row0 = s * _ROWS_PER_TILE
    for off, n in _SLAB_CHUNKS:
        pltpu.sync_copy(rows_v.at[pl.ds(0, n)],
                        acc_sh.at[pl.ds(row0 + off, n)])
    plsc.subcore_barrier()

    def _round(t, carry):
        base = pl.multiple_of(wid * _E_PER_W + t * _GCHUNK, 8)
        pltpu.sync_copy(src_hbm.at[pl.ds(base, _GCHUNK)], src_v)
        for q in range(_NSUB):
            pltpu.sync_copy(dst_hbm.at[pl.ds(base + q * _CHUNK, _CHUNK)],
                            dst_v.at[q])
        pltpu.async_copy(h_hbm.at[src_v], rows_v, gsem).wait()
        for q in range(_NSUB):
            pltpu.sync_copy(rows_v.at[pl.ds(q * _CHUNK, _CHUNK)],
                            acc_sh.at[dst_v.at[q]], add=True)
        return carry

    lax.fori_loop(0, _N_ROUNDS, _round, None)
    plsc.subcore_barrier()

    for off, n in _SLAB_CHUNKS:
        r = row0 + off
        pltpu.sync_copy(acc_sh.at[pl.ds(r, n)], rows_v.at[pl.ds(0, n)])
        pltpu.sync_copy(rows_v.at[pl.ds(0, n)], acc_out.at[c, pl.ds(r, n)])


@functools.cache
def _sc_segsum_call():
    return pl.kernel(
        _sc_body,
        out_type=jax.ShapeDtypeStruct((_NC, _N_PAD, _D), jnp.float32),
        mesh=plsc.VectorSubcoreMesh(core_axis_name="c", subcore_axis_name="s",
                                    num_cores=_NC, num_subcores=_NS),
        scratch_types=[
            pltpu.VMEM((_GCHUNK,), jnp.int32),
            pltpu.VMEM((_NSUB, _CHUNK), jnp.int32),
            pltpu.VMEM((_GCHUNK, _D), jnp.float32),
            pltpu.VMEM_SHARED((_N_PAD, _D), jnp.float32),
            pltpu.SemaphoreType.DMA,
        ],
    )


def _tc_body(do_relu, acc0, acc1, deg0, deg1, h, wrelT, b, wrootT, out_ref):
    summed = acc0[...] + acc1[...]
    deg = deg0[...] + deg1[...]
    degc = jnp.maximum(deg[:, 0:1], 1.0)
    mean = summed / degc
    r = (jnp.dot(mean, wrelT[...], preferred_element_type=jnp.float32)
         + jnp.dot(h[...], wrootT[...], preferred_element_type=jnp.float32)
         + b[...])
    if do_relu:
        r = jnp.maximum(r, 0.0)
    out_ref[...] = r


_TC_BLK = 1264


def _tc_call(do_relu, acc0, acc1, deg0, deg1, h, wrelT, b, wrootT):
    grid = (_N_PAD // _TC_BLK,)
    return pl.pallas_call(
        functools.partial(_tc_body, do_relu),
        grid=grid,
        in_specs=[
            pl.BlockSpec((_TC_BLK, _D), lambda i: (i, 0)),
            pl.BlockSpec((_TC_BLK, _D), lambda i: (i, 0)),
            pl.BlockSpec((_TC_BLK, _D), lambda i: (i, 0)),
            pl.BlockSpec((_TC_BLK, _D), lambda i: (i, 0)),
            pl.BlockSpec((_TC_BLK, _D), lambda i: (i, 0)),
            pl.BlockSpec((_D, _D), lambda i: (0, 0)),
            pl.BlockSpec((1, _D), lambda i: (0, 0)),
            pl.BlockSpec((_D, _D), lambda i: (0, 0)),
        ],
        out_specs=pl.BlockSpec((_TC_BLK, _D), lambda i: (i, 0)),
        out_shape=jax.ShapeDtypeStruct((_N_PAD, _D), jnp.float32),
    )(acc0, acc1, deg0, deg1, h, wrelT, b, wrootT)


def kernel(x, edge_index, W_rel0, b_rel0, W_root0, W_rel1, b_rel1, W_root1):
    x_pad = jnp.zeros((_N_PAD, _D), jnp.float32).at[:_N].set(x)
    pad_idx = jnp.full((_E_PAD - _E,), _N, jnp.int32)
    src = jnp.concatenate([edge_index[0], pad_idx])
    dst = jnp.concatenate([edge_index[1], pad_idx])

    deg = _sc_degree_call()(dst)
    acc_a = _sc_segsum_call()(x_pad, src, dst)
    h1 = _tc_call(True, acc_a[0], acc_a[1], deg[0], deg[1], x_pad,
                  W_rel0.T, b_rel0.reshape(1, _D), W_root0.T)
    acc_b = _sc_segsum_call()(h1, src, dst)
    out = _tc_call(False, acc_b[0], acc_b[1], deg[0], deg[1], h1,
                   W_rel1.T, b_rel1.reshape(1, _D), W_root1.T)
    return out[:_N]


# speedup vs baseline: 2.3225x; 2.3225x over previous
import functools

import jax
import jax.numpy as jnp
from jax import lax
from jax.experimental import pallas as pl
from jax.experimental.pallas import tpu as pltpu
from jax.experimental.pallas import tpu_sc as plsc

_N, _E, _D = 10000, 320000, 128
_NC, _NS = 2, 16
_NW = _NC * _NS
_CHUNK = 128
_GCHUNK = 256
_NSUB = _GCHUNK // _CHUNK
_N_PAD = 10112
_E_PER_W = 10240
_E_PAD = _E_PER_W * _NW
_N_ROUNDS = _E_PER_W // _GCHUNK
_ROWS_PER_TILE = _N_PAD // _NS
_SLAB_CHUNKS = []
_off = 0
while _off < _ROWS_PER_TILE:
    _n = min(_GCHUNK, _ROWS_PER_TILE - _off)
    _SLAB_CHUNKS.append((_off, _n))
    _off += _n


def _deg_body(dst_hbm, deg_out, dst_v, ones_v, deg_sh):
    c = lax.axis_index("c")
    s = lax.axis_index("s")
    wid = c * _NS + s

    zeros16 = jnp.zeros((16,), jnp.float32)
    ones16 = jnp.ones((16,), jnp.float32)

    def _fill_zeros(r, carry):
        for j in range(_D // 16):
            ones_v[r, pl.ds(j * 16, 16)] = zeros16
        return carry

    lax.fori_loop(0, _CHUNK, _fill_zeros, None)

    row0 = s * _ROWS_PER_TILE
    for off, n in _SLAB_CHUNKS:
        for o2 in range(0, n, _CHUNK):
            m = min(_CHUNK, n - o2)
            pltpu.sync_copy(ones_v.at[pl.ds(0, m)],
                            deg_sh.at[pl.ds(row0 + off + o2, m)])
    plsc.subcore_barrier()

    def _fill_ones(r, carry):
        for j in range(_D // 16):
            ones_v[r, pl.ds(j * 16, 16)] = ones16
        return carry

    lax.fori_loop(0, _CHUNK, _fill_ones, None)

    def _round(t, carry):
        base = pl.multiple_of(wid * _E_PER_W + t * _GCHUNK, 8)
        for q in range(_NSUB):
            pltpu.sync_copy(dst_hbm.at[pl.ds(base + q * _CHUNK, _CHUNK)],
                            dst_v.at[q])
        for q in range(_NSUB):
            pltpu.sync_copy(ones_v, deg_sh.at[dst_v.at[q]], add=True)
        return carry

    lax.fori_loop(0, _N_ROUNDS, _round, None)
    plsc.subcore_barrier()

    for off, n in _SLAB_CHUNKS:
        for o2 in range(0, n, _CHUNK):
            m = min(_CHUNK, n - o2)
            r = row0 + off + o2
            pltpu.sync_copy(deg_sh.at[pl.ds(r, m)], ones_v.at[pl.ds(0, m)])
            pltpu.sync_copy(ones_v.at[pl.ds(0, m)], deg_out.at[c, pl.ds(r, m)])


@functools.cache
def _sc_degree_call():
    return pl.kernel(
        _deg_body,
        out_type=jax.ShapeDtypeStruct((_NC, _N_PAD, _D), jnp.float32),
        mesh=plsc.VectorSubcoreMesh(core_axis_name="c", subcore_axis_name="s",
                                    num_cores=_NC, num_subcores=_NS),
        scratch_types=[
            pltpu.VMEM((_NSUB, _CHUNK), jnp.int32),
            pltpu.VMEM((_CHUNK, _D), jnp.float32),
            pltpu.VMEM_SHARED((_N_PAD, _D), jnp.float32),
        ],
    )


def _sc_body(h_hbm, src_hbm, dst_hbm, acc_out,
             src_v, dst_v, rows_v, acc_sh, gsem):
    c = lax.axis_index("c")
    s = lax.axis_index("s")
    wid = c * _NS + s

    zeros16 = jnp.zeros((16,), jnp.float32)

    def _fill_zeros(r, carry):
        for j in range(_D // 16):
            rows_v[r, pl.ds(j * 16, 16)] = zeros16
        return carry

    lax.fori_loop(0, _GCHUNK, _fill_zeros, None)

    row0 = s * _ROWS_PER_TILE
    for off, n in _SLAB_CHUNKS:
        pltpu.sync_copy(rows_v.at[pl.ds(0, n)],
                        acc_sh.at[pl.ds(row0 + off, n)])
    plsc.subcore_barrier()

    def _round(t, carry):
        base = pl.multiple_of(wid * _E_PER_W + t * _GCHUNK, 8)
        pltpu.sync_copy(src_hbm.at[pl.ds(base, _GCHUNK)], src_v)
        for q in range(_NSUB):
            pltpu.sync_copy(dst_hbm.at[pl.ds(base + q * _CHUNK, _CHUNK)],
                            dst_v.at[q])
        pltpu.async_copy(h_hbm.at[src_v], rows_v, gsem).wait()
        for q in range(_NSUB):
            pltpu.sync_copy(rows_v.at[pl.ds(q * _CHUNK, _CHUNK)],
                            acc_sh.at[dst_v.at[q]], add=True)
        return carry

    lax.fori_loop(0, _N_ROUNDS, _round, None)
    plsc.subcore_barrier()

    for off, n in _SLAB_CHUNKS:
        r = row0 + off
        pltpu.sync_copy(acc_sh.at[pl.ds(r, n)], rows_v.at[pl.ds(0, n)])
        pltpu.sync_copy(rows_v.at[pl.ds(0, n)], acc_out.at[c, pl.ds(r, n)])


@functools.cache
def _sc_segsum_call():
    return pl.kernel(
        _sc_body,
        out_type=jax.ShapeDtypeStruct((_NC, _N_PAD, _D), jnp.float32),
        mesh=plsc.VectorSubcoreMesh(core_axis_name="c", subcore_axis_name="s",
                                    num_cores=_NC, num_subcores=_NS),
        scratch_types=[
            pltpu.VMEM((_GCHUNK,), jnp.int32),
            pltpu.VMEM((_NSUB, _CHUNK), jnp.int32),
            pltpu.VMEM((_GCHUNK, _D), jnp.float32),
            pltpu.VMEM_SHARED((_N_PAD, _D), jnp.float32),
            pltpu.SemaphoreType.DMA,
        ],
    )


def _tc_body(do_relu, acc0, acc1, deg0, deg1, h, wrelT, b, wrootT, out_ref):
    summed = acc0[...] + acc1[...]
    deg = deg0[...] + deg1[...]
    degc = jnp.maximum(deg[:, 0:1], 1.0)
    mean = summed / degc
    r = (jnp.dot(mean, wrelT[...], preferred_element_type=jnp.float32)
         + jnp.dot(h[...], wrootT[...], preferred_element_type=jnp.float32)
         + b[...])
    if do_relu:
        r = jnp.maximum(r, 0.0)
    out_ref[...] = r


_TC_BLK = 1264


def _tc_call(do_relu, acc0, acc1, deg0, deg1, h, wrelT, b, wrootT):
    grid = (_N_PAD // _TC_BLK,)
    return pl.pallas_call(
        functools.partial(_tc_body, do_relu),
        grid=grid,
        in_specs=[
            pl.BlockSpec((_TC_BLK, _D), lambda i: (i, 0)),
            pl.BlockSpec((_TC_BLK, _D), lambda i: (i, 0)),
            pl.BlockSpec((_TC_BLK, _D), lambda i: (i, 0)),
            pl.BlockSpec((_TC_BLK, _D), lambda i: (i, 0)),
            pl.BlockSpec((_TC_BLK, _D), lambda i: (i, 0)),
            pl.BlockSpec((_D, _D), lambda i: (0, 0)),
            pl.BlockSpec((1, _D), lambda i: (0, 0)),
            pl.BlockSpec((_D, _D), lambda i: (0, 0)),
        ],
        out_specs=pl.BlockSpec((_TC_BLK, _D), lambda i: (i, 0)),
        out_shape=jax.ShapeDtypeStruct((_N_PAD, _D), jnp.float32),
    )(acc0, acc1, deg0, deg1, h, wrelT, b, wrootT)


def kernel(x, edge_index, W_rel0, b_rel0, W_root0, W_rel1, b_rel1, W_root1):
    x_pad = jnp.zeros((_N_PAD, _D), jnp.float32).at[:_N].set(x)
    pad_idx = _N + (jnp.arange(_E_PAD - _E, dtype=jnp.int32) % (_N_PAD - _N))
    src = jnp.concatenate([edge_index[0], pad_idx])
    dst = jnp.concatenate([edge_index[1], pad_idx])

    deg = _sc_degree_call()(dst)
    acc_a = _sc_segsum_call()(x_pad, src, dst)
    h1 = _tc_call(True, acc_a[0], acc_a[1], deg[0], deg[1], x_pad,
                  W_rel0.T, b_rel0.reshape(1, _D), W_root0.T)
    acc_b = _sc_segsum_call()(h1, src, dst)
    out = _tc_call(False, acc_b[0], acc_b[1], deg[0], deg[1], h1,
                   W_rel1.T, b_rel1.reshape(1, _D), W_root1.T)
    return out[:_N]
